# Initial kernel scaffold; baseline (speedup 1.0000x reference)
import jax
import jax.numpy as jnp
from jax.experimental import pallas as pl


def kernel(x, batch, W, b):
    raise NotImplementedError("write your pallas kernel here")


# speedup vs baseline: 4.2595x; 4.2595x over previous
import jax
import jax.numpy as jnp
from jax import lax
from jax.experimental import pallas as pl
from jax.experimental.pallas import tpu as pltpu
from jax.experimental.pallas import tpu_sc as plsc

N = 1600000
D = 15
S = 4096
OUT = 5

NC = 2
NS = 16
NW = NC * NS
LANES = 16
ROWS_PER_TILE = N // NW
CHUNK = 2000
NCHUNKS = ROWS_PER_TILE // CHUNK


def _sc_body(x_hbm, batch_hbm, out_hbm, xbuf, bbuf, acc):
    wid = lax.axis_index("s") * NC + lax.axis_index("c")
    base_row = wid * ROWS_PER_TILE
    iota = lax.iota(jnp.int32, LANES)
    cols = jnp.where(iota < D, iota, D - 1)
    lane_base = iota * S
    lane_is_feat = iota < D
    ones = jnp.ones((LANES,), jnp.float32)
    zeros = jnp.zeros((LANES,), jnp.float32)

    def zero_body(j, _):
        plsc.store_scatter(acc, [j * LANES + iota], zeros)
        return 0
    lax.fori_loop(0, LANES * S // LANES, zero_body, 0)

    def chunk_body(c, _):
        r0 = base_row + c * CHUNK
        pltpu.sync_copy(x_hbm.at[pl.ds(r0 * D, CHUNK * D)], xbuf)
        pltpu.sync_copy(batch_hbm.at[pl.ds(r0, CHUNK)], bbuf)

        def group_body(g, _):
            bvec = bbuf[pl.ds(g * LANES, LANES)]
            for k in range(LANES):
                r = g * LANES + k
                row = plsc.load_gather(xbuf, [jnp.full((LANES,), r * D, jnp.int32) + cols])
                vals = jnp.where(lane_is_feat, row, ones)
                plsc.addupdate_scatter(
                    acc, [lane_base + jnp.full((LANES,), bvec[k], jnp.int32)], vals)
            return 0
        lax.fori_loop(0, CHUNK // LANES, group_body, 0)
        return 0
    lax.fori_loop(0, NCHUNKS, chunk_body, 0)

    pltpu.sync_copy(acc, out_hbm.at[wid])


_sc_segment_sum = pl.kernel(
    _sc_body,
    out_type=jax.ShapeDtypeStruct((NW, LANES * S), jnp.float32),
    mesh=plsc.VectorSubcoreMesh(core_axis_name="c", subcore_axis_name="s"),
    compiler_params=pltpu.CompilerParams(needs_layout_passes=False),
    scratch_types=[
        pltpu.VMEM((CHUNK * D,), jnp.float32),
        pltpu.VMEM((CHUNK,), jnp.int32),
        pltpu.VMEM((LANES * S,), jnp.float32),
    ],
)


def _tc_tail_body(p_ref, w_ref, b_ref, o_ref):
    s = jnp.sum(p_ref[...], axis=0)
    counts = s[D, :]
    mean = s[:D, :] / jnp.maximum(counts, 1.0)[None, :]
    out = lax.dot_general(mean, w_ref[...], (((0,), (1,)), ((), ())),
                          preferred_element_type=jnp.float32)
    o_ref[...] = jnp.where(counts[:, None] > 0, out + b_ref[...][None, :], 0.0)


_tc_tail = pl.pallas_call(
    _tc_tail_body,
    out_shape=jax.ShapeDtypeStruct((S, OUT), jnp.float32),
)


def kernel(x, batch, W, b):
    partials = _sc_segment_sum(x.reshape(N * D), batch.astype(jnp.int32))
    return _tc_tail(partials.reshape(NW, LANES, S), W, b)


# speedup vs baseline: 4.7758x; 1.1212x over previous
import jax
import jax.numpy as jnp
from jax import lax
from jax.experimental import pallas as pl
from jax.experimental.pallas import tpu as pltpu
from jax.experimental.pallas import tpu_sc as plsc

N = 1600000
D = 15
S = 4096
OUT = 5

NC = 2
NS = 16
NW = NC * NS
LANES = 16
ROWS_PER_TILE = N // NW
CHUNK = 2000
NCHUNKS = ROWS_PER_TILE // CHUNK


def _sc_body(x_hbm, batch_hbm, out_hbm, xbuf, bbuf, acc):
    wid = lax.axis_index("s") * NC + lax.axis_index("c")
    base_row = wid * ROWS_PER_TILE
    iota = lax.iota(jnp.int32, LANES)
    cols = jnp.where(iota < D, iota, D - 1)
    lane_base = iota * S
    lane_is_feat = iota < D
    ones = jnp.ones((LANES,), jnp.float32)
    zeros = jnp.zeros((LANES,), jnp.float32)

    @plsc.parallel_loop(0, LANES * S, step=LANES, unroll=4)
    def _zero(j):
        acc[pl.ds(j, LANES)] = zeros

    def chunk_body(c, _):
        r0 = base_row + c * CHUNK
        pltpu.sync_copy(x_hbm.at[pl.ds(r0 * D, CHUNK * D)], xbuf.at[pl.ds(0, CHUNK * D)])
        pltpu.sync_copy(batch_hbm.at[pl.ds(r0, CHUNK)], bbuf)

        @plsc.parallel_loop(0, CHUNK // LANES, unroll=2)
        def _groups(g):
            g0 = g * LANES
            bvec = bbuf[pl.ds(g0, LANES)]
            for k in range(LANES):
                row = xbuf[pl.ds((g0 + k) * D, LANES)]
                vals = jnp.where(lane_is_feat, row, ones)
                plsc.addupdate_scatter(
                    acc, [lane_base + jnp.full((LANES,), bvec[k], jnp.int32)], vals)
        return 0
    lax.fori_loop(0, NCHUNKS, chunk_body, 0)

    pltpu.sync_copy(acc, out_hbm.at[wid])


_sc_segment_sum = pl.kernel(
    _sc_body,
    out_type=jax.ShapeDtypeStruct((NW, LANES * S), jnp.float32),
    mesh=plsc.VectorSubcoreMesh(core_axis_name="c", subcore_axis_name="s"),
    compiler_params=pltpu.CompilerParams(needs_layout_passes=False),
    scratch_types=[
        pltpu.VMEM((CHUNK * D + LANES,), jnp.float32),
        pltpu.VMEM((CHUNK,), jnp.int32),
        pltpu.VMEM((LANES * S,), jnp.float32),
    ],
)


def _tc_tail_body(p_ref, w_ref, b_ref, o_ref):
    s = jnp.sum(p_ref[...], axis=0)
    counts = s[D, :]
    mean = s[:D, :] / jnp.maximum(counts, 1.0)[None, :]
    out = lax.dot_general(mean, w_ref[...], (((0,), (1,)), ((), ())),
                          preferred_element_type=jnp.float32)
    o_ref[...] = jnp.where(counts[:, None] > 0, out + b_ref[...][None, :], 0.0)


_tc_tail = pl.pallas_call(
    _tc_tail_body,
    out_shape=jax.ShapeDtypeStruct((S, OUT), jnp.float32),
)


def kernel(x, batch, W, b):
    partials = _sc_segment_sum(x.reshape(N * D), batch.astype(jnp.int32))
    return _tc_tail(partials.reshape(NW, LANES, S), W, b)


# speedup vs baseline: 6.2624x; 1.3113x over previous
import jax
import jax.numpy as jnp
from jax import lax
from jax.experimental import pallas as pl
from jax.experimental.pallas import tpu as pltpu
from jax.experimental.pallas import tpu_sc as plsc

N = 1600000
D = 15
S = 4096
OUT = 5

NC = 2
NS = 16
NW = NC * NS
LANES = 16
ROWS_PER_TILE = N // NW
CHUNK = 2000
NCHUNKS = ROWS_PER_TILE // CHUNK


def _sc_body(x_hbm, batch_hbm, out_hbm, xbuf, bbuf, acc):
    wid = lax.axis_index("s") * NC + lax.axis_index("c")
    base_row = wid * ROWS_PER_TILE
    iota = lax.iota(jnp.int32, LANES)
    lane_is_feat = iota < D
    ones = jnp.ones((LANES,), jnp.float32)
    zeros = jnp.zeros((LANES,), jnp.float32)

    @plsc.parallel_loop(0, LANES * S, step=LANES, unroll=4)
    def _zero(j):
        acc[pl.ds(j, LANES)] = zeros

    def chunk_body(c, _):
        r0 = base_row + c * CHUNK
        pltpu.sync_copy(x_hbm.at[pl.ds(r0 * D, CHUNK * D)], xbuf.at[pl.ds(0, CHUNK * D)])
        pltpu.sync_copy(batch_hbm.at[pl.ds(r0, CHUNK)], bbuf)

        @plsc.parallel_loop(0, CHUNK // LANES, unroll=2)
        def _groups(g):
            g0 = g * LANES
            bvec = bbuf[pl.ds(g0, LANES)] * LANES
            for k in range(LANES):
                row = xbuf[pl.ds((g0 + k) * D, LANES)]
                vals = jnp.where(lane_is_feat, row, ones)
                plsc.addupdate_scatter(
                    acc, [jnp.full((LANES,), bvec[k], jnp.int32) + iota], vals)
        return 0
    lax.fori_loop(0, NCHUNKS, chunk_body, 0)

    pltpu.sync_copy(acc, out_hbm.at[wid])


_sc_segment_sum = pl.kernel(
    _sc_body,
    out_type=jax.ShapeDtypeStruct((NW, S * LANES), jnp.float32),
    mesh=plsc.VectorSubcoreMesh(core_axis_name="c", subcore_axis_name="s"),
    compiler_params=pltpu.CompilerParams(needs_layout_passes=False),
    scratch_types=[
        pltpu.VMEM((CHUNK * D + LANES,), jnp.float32),
        pltpu.VMEM((CHUNK,), jnp.int32),
        pltpu.VMEM((LANES * S,), jnp.float32),
    ],
)


def _tc_tail_body(p_ref, w_ref, b_ref, o_ref, acc_ref):
    i = pl.program_id(0)

    @pl.when(i == 0)
    def _init():
        acc_ref[...] = p_ref[0]

    @pl.when(i > 0)
    def _accum():
        acc_ref[...] += p_ref[0]

    @pl.when(i == NW - 1)
    def _finish():
        s = acc_ref[...]
        counts = s[:, D]
        mean = s[:, :D] / jnp.maximum(counts, 1.0)[:, None]
        out = lax.dot_general(mean, w_ref[...], (((1,), (1,)), ((), ())),
                              preferred_element_type=jnp.float32)
        o_ref[...] = jnp.where(counts[:, None] > 0, out + b_ref[...][None, :], 0.0)


_tc_tail = pl.pallas_call(
    _tc_tail_body,
    grid=(NW,),
    in_specs=[
        pl.BlockSpec((1, S, LANES), lambda i: (i, 0, 0)),
        pl.BlockSpec((OUT, D), lambda i: (0, 0)),
        pl.BlockSpec((OUT,), lambda i: (0,)),
    ],
    out_specs=pl.BlockSpec((S, OUT), lambda i: (0, 0)),
    scratch_shapes=[pltpu.VMEM((S, LANES), jnp.float32)],
    out_shape=jax.ShapeDtypeStruct((S, OUT), jnp.float32),
)


def kernel(x, batch, W, b):
    partials = _sc_segment_sum(x.reshape(N * D), batch.astype(jnp.int32))
    return _tc_tail(partials.reshape(NW, S, LANES), W, b)
